# baseline (speedup 1.0000x reference)
import jax
import jax.numpy as jnp
import numpy as np
from jax.experimental import pallas as pl

_PARTS = 8
_MU = 0.0001
_TWO_PI = 2.0 * np.pi
_N = 64
_F = 16


def _build_select_a() -> np.ndarray:
    s = np.zeros((_N * _F, 3 * _N), dtype=np.float32)
    for n in range(_N):
        s[_F * n + 0, 0 * _N + n] = 1.0
        s[_F * n + 1, 1 * _N + n] = 1.0
        s[_F * n: _F * (n + 1), 2 * _N + n] = 1.0
    return s


def _build_select_b() -> np.ndarray:
    s = np.zeros((_N * _F, 2 * _N), dtype=np.float32)
    for n in range(_N):
        s[_F * n + 14, 0 * _N + n] = 1.0
        s[_F * n + 15, 1 * _N + n] = 1.0
    return s


def _build_reduce() -> np.ndarray:
    p = np.zeros((32 * _N, 32), dtype=np.float32)
    for a in range(_PARTS):
        for q in range(4):
            k = a * 4 + q
            p[k * _N: (k + 1) * _N, q * _PARTS + a] = 1.0
    return p


def _dot(a, b, precision=None):
    return jax.lax.dot_general(
        a, b, (((1,), (0,)), ((), ())),
        precision=precision,
        preferred_element_type=jnp.float32)


def _sc_kernel(tr_ref, nt_ref, sela_ref, selb_ref, red_ref, spd_ref, dst_ref,
               drc_ref, fdir_ref):
    x = nt_ref[...]
    t = tr_ref[...]

    feat = _dot(x, sela_ref[...])
    fx = feat[:, 0 * _N:1 * _N]
    fy = feat[:, 1 * _N:2 * _N]
    nei_sum = feat[:, 2 * _N:3 * _N]

    ll = _dot(x, selb_ref[...], jax.lax.Precision.HIGHEST)
    lx = ll[:, 0 * _N:1 * _N]
    ly = ll[:, 1 * _N:2 * _N]

    tx0 = t[:, 0:1]
    ty0 = t[:, 1:2]
    tx1 = t[:, 14:15]
    ty1 = t[:, 15:16]

    vx = lx - fx
    vy = ly - fy
    nei_len = jnp.sqrt(vx * vx + vy * vy)
    ovx = tx1 - tx0
    ovy = ty1 - ty0
    obs_len = jnp.sqrt(ovx * ovx + ovy * ovy)
    f_speed = (nei_len + _MU) / (obs_len + _MU)

    px = lx - tx1
    py = ly - ty1
    f_dist = jnp.sqrt(px * px + py * py)
    f_dir = jnp.arctan2(py, px)
    f_dir = jnp.mod(f_dir, _TWO_PI)

    ang = (f_dir / (_TWO_PI / _PARTS)).astype(jnp.int32)
    ang = jnp.where(nei_sum != 0.0, ang, -1)

    fdir_ref[...] = f_dir

    one = jnp.ones_like(f_dir)
    vals4 = jnp.concatenate([one, f_speed, f_dist, f_dir], axis=1)
    ang4 = jnp.concatenate([ang, ang, ang, ang], axis=1)
    zero4 = jnp.zeros_like(vals4)
    masked = jnp.concatenate(
        [jnp.where(ang4 == a, vals4, zero4) for a in range(_PARTS)],
        axis=1)
    sums = _dot(masked, red_ref[...])

    n8 = sums[:, 0:8] + 0.0001
    spd_ref[...] = sums[:, 8:16] / n8
    dst_ref[...] = sums[:, 16:24] / n8
    drc_ref[...] = sums[:, 24:32] / n8


def kernel(trajs, nei_trajs):
    B = trajs.shape[0]
    tr = trajs.reshape(B, 16)
    nt = nei_trajs.reshape(B, _N * _F)
    sela = jnp.asarray(_build_select_a())
    selb = jnp.asarray(_build_select_b())
    red = jnp.asarray(_build_reduce())
    BB = 512
    grid = (B // BB,)
    spd, dst, drc, f_dir = pl.pallas_call(
        _sc_kernel,
        grid=grid,
        in_specs=[
            pl.BlockSpec((BB, 16), lambda i: (i, 0)),
            pl.BlockSpec((BB, _N * _F), lambda i: (i, 0)),
            pl.BlockSpec((_N * _F, 3 * _N), lambda i: (0, 0)),
            pl.BlockSpec((_N * _F, 2 * _N), lambda i: (0, 0)),
            pl.BlockSpec((32 * _N, 32), lambda i: (0, 0)),
        ],
        out_specs=[
            pl.BlockSpec((BB, 8), lambda i: (i, 0)),
            pl.BlockSpec((BB, 8), lambda i: (i, 0)),
            pl.BlockSpec((BB, 8), lambda i: (i, 0)),
            pl.BlockSpec((BB, _N), lambda i: (i, 0)),
        ],
        out_shape=[
            jax.ShapeDtypeStruct((B, 8), jnp.float32),
            jax.ShapeDtypeStruct((B, 8), jnp.float32),
            jax.ShapeDtypeStruct((B, 8), jnp.float32),
            jax.ShapeDtypeStruct((B, _N), jnp.float32),
        ],
    )(tr, nt, sela, selb, red)
    return jnp.stack([spd, dst, drc], axis=2), f_dir


# speedup vs baseline: 1.0869x; 1.0869x over previous
import jax
import jax.numpy as jnp
import numpy as np
from jax.experimental import pallas as pl

_PARTS = 8
_MU = 0.0001
_TWO_PI = 2.0 * np.pi
_N = 64
_F = 16


def _build_select_a() -> np.ndarray:
    s = np.zeros((_N * _F, 3 * _N), dtype=np.float32)
    for n in range(_N):
        s[_F * n + 0, 0 * _N + n] = 1.0
        s[_F * n + 1, 1 * _N + n] = 1.0
        s[_F * n: _F * (n + 1), 2 * _N + n] = 1.0
    return s


def _build_select_b() -> np.ndarray:
    s = np.zeros((_N * _F, 2 * _N), dtype=np.float32)
    for n in range(_N):
        s[_F * n + 14, 0 * _N + n] = 1.0
        s[_F * n + 15, 1 * _N + n] = 1.0
    return s


def _build_reduce() -> np.ndarray:
    p = np.zeros((32 * _N, 32), dtype=np.float32)
    for a in range(_PARTS):
        for q in range(4):
            k = a * 4 + q
            p[k * _N: (k + 1) * _N, q * _PARTS + a] = 1.0
    return p


def _dot(a, b):
    return jax.lax.dot_general(
        a, b, (((1,), (0,)), ((), ())),
        preferred_element_type=jnp.float32)


def _sc_kernel(tr_ref, nt_ref, sela_ref, selb_ref, red_ref, spd_ref, dst_ref,
               drc_ref, fdir_ref):
    x = nt_ref[...]
    t = tr_ref[...]

    feat = _dot(x, sela_ref[...])
    fx = feat[:, 0 * _N:1 * _N]
    fy = feat[:, 1 * _N:2 * _N]
    nei_sum = feat[:, 2 * _N:3 * _N]

    selb = selb_ref[...]
    x_hi = x.astype(jnp.bfloat16).astype(jnp.float32)
    r = x - x_hi
    x_mid = r.astype(jnp.bfloat16).astype(jnp.float32)
    x_lo = r - x_mid
    ll = (_dot(x_hi, selb) + _dot(x_mid, selb)) + _dot(x_lo, selb)
    lx = ll[:, 0 * _N:1 * _N]
    ly = ll[:, 1 * _N:2 * _N]

    tx0 = t[:, 0:1]
    ty0 = t[:, 1:2]
    tx1 = t[:, 14:15]
    ty1 = t[:, 15:16]

    vx = lx - fx
    vy = ly - fy
    nei_len = jnp.sqrt(vx * vx + vy * vy)
    ovx = tx1 - tx0
    ovy = ty1 - ty0
    obs_len = jnp.sqrt(ovx * ovx + ovy * ovy)
    f_speed = (nei_len + _MU) / (obs_len + _MU)

    px = lx - tx1
    py = ly - ty1
    f_dist = jnp.sqrt(px * px + py * py)
    f_dir = jnp.arctan2(py, px)
    f_dir = jnp.mod(f_dir, _TWO_PI)

    ang = (f_dir / (_TWO_PI / _PARTS)).astype(jnp.int32)
    ang = jnp.where(nei_sum != 0.0, ang, -1)

    fdir_ref[...] = f_dir

    one = jnp.ones_like(f_dir)
    vals4 = jnp.concatenate([one, f_speed, f_dist, f_dir], axis=1)
    ang4 = jnp.concatenate([ang, ang, ang, ang], axis=1)
    zero4 = jnp.zeros_like(vals4)
    masked = jnp.concatenate(
        [jnp.where(ang4 == a, vals4, zero4) for a in range(_PARTS)],
        axis=1)
    sums = _dot(masked, red_ref[...])

    n8 = sums[:, 0:8] + 0.0001
    spd_ref[...] = sums[:, 8:16] / n8
    dst_ref[...] = sums[:, 16:24] / n8
    drc_ref[...] = sums[:, 24:32] / n8


def kernel(trajs, nei_trajs):
    B = trajs.shape[0]
    tr = trajs.reshape(B, 16)
    nt = nei_trajs.reshape(B, _N * _F)
    sela = jnp.asarray(_build_select_a())
    selb = jnp.asarray(_build_select_b())
    red = jnp.asarray(_build_reduce())
    BB = 512
    grid = (B // BB,)
    spd, dst, drc, f_dir = pl.pallas_call(
        _sc_kernel,
        grid=grid,
        in_specs=[
            pl.BlockSpec((BB, 16), lambda i: (i, 0)),
            pl.BlockSpec((BB, _N * _F), lambda i: (i, 0)),
            pl.BlockSpec((_N * _F, 3 * _N), lambda i: (0, 0)),
            pl.BlockSpec((_N * _F, 2 * _N), lambda i: (0, 0)),
            pl.BlockSpec((32 * _N, 32), lambda i: (0, 0)),
        ],
        out_specs=[
            pl.BlockSpec((BB, 8), lambda i: (i, 0)),
            pl.BlockSpec((BB, 8), lambda i: (i, 0)),
            pl.BlockSpec((BB, 8), lambda i: (i, 0)),
            pl.BlockSpec((BB, _N), lambda i: (i, 0)),
        ],
        out_shape=[
            jax.ShapeDtypeStruct((B, 8), jnp.float32),
            jax.ShapeDtypeStruct((B, 8), jnp.float32),
            jax.ShapeDtypeStruct((B, 8), jnp.float32),
            jax.ShapeDtypeStruct((B, _N), jnp.float32),
        ],
    )(tr, nt, sela, selb, red)
    return jnp.stack([spd, dst, drc], axis=2), f_dir
